# baseline (speedup 1.0000x reference)
import functools

import jax
import jax.numpy as jnp
from jax import lax
from jax.experimental import pallas as pl
from jax.experimental.pallas import tpu as pltpu
from jax.experimental.pallas import tpu_sc as plsc

VOCAB = 100000
EMBED_DIM = 64
OUTPUT_DIM = 2
BATCH = 4096
SEQ = 200

DP = 16
NC, NS = 2, 16
NW = NC * NS
B_PER_W = BATCH // NW
CB = 8
NCH = B_PER_W // CB
GW = 100
GPC = CB * SEQ // GW
ROWS_PER_CHUNK = CB * SEQ
IDX_ROWS_PER_W = B_PER_W * SEQ // GW


def _proj_body(emb_ref, w_ref, b_ref, out_ref):
    acc = jnp.dot(emb_ref[...], w_ref[...].T, preferred_element_type=jnp.float32)
    out_ref[...] = (acc + b_ref[...]) * (1.0 / SEQ)


def _project_table(emb_table, w_pad, b_pad):
    blk = 10000
    grid = VOCAB // blk
    return pl.pallas_call(
        _proj_body,
        grid=(grid,),
        in_specs=[
            pl.BlockSpec((blk, EMBED_DIM), lambda i: (i, 0)),
            pl.BlockSpec((DP, EMBED_DIM), lambda i: (0, 0)),
            pl.BlockSpec((1, DP), lambda i: (0, 0)),
        ],
        out_specs=pl.BlockSpec((blk, DP), lambda i: (i, 0)),
        out_shape=jax.ShapeDtypeStruct((VOCAB, DP), jnp.float32),
    )(emb_table, w_pad, b_pad)


def _sc_body(proj_hbm, text_hbm, out_hbm, idx_v, rows_v, out_v, sem):
    wid = lax.axis_index("s") * NC + lax.axis_index("c")

    pltpu.sync_copy(text_hbm.at[pl.ds(wid * IDX_ROWS_PER_W, IDX_ROWS_PER_W)], idx_v)

    def fire(c, slot):
        def issue(k, carry):
            pltpu.async_copy(
                proj_hbm.at[idx_v.at[c * GPC + k]],
                rows_v.at[slot, pl.ds(k * GW, GW)],
                sem,
            )
            return carry

        lax.fori_loop(0, GPC, issue, 0)

    def drain(slot):
        pltpu.make_async_copy(
            proj_hbm.at[pl.ds(0, ROWS_PER_CHUNK)],
            rows_v.at[slot],
            sem,
        ).wait()

    def chunk_body(c, carry):
        slot = lax.rem(c, 2)
        drain(slot)

        @pl.when(c + 1 < NCH)
        def _():
            fire(c + 1, 1 - slot)

        for i in range(CB):
            base = i * SEQ

            def red_body(j, accs):
                o = base + 8 * j
                return tuple(accs[t] + rows_v[slot, o + t] for t in range(8))

            z = jnp.zeros((DP,), jnp.float32)
            accs = lax.fori_loop(0, SEQ // 8, red_body, (z,) * 8)
            s0 = (accs[0] + accs[1]) + (accs[2] + accs[3])
            s1 = (accs[4] + accs[5]) + (accs[6] + accs[7])
            out_v[i] = s0 + s1
        pltpu.sync_copy(out_v, out_hbm.at[pl.ds(wid * B_PER_W + c * CB, CB)])
        return carry

    fire(0, 0)
    lax.fori_loop(0, NCH, chunk_body, 0)


@functools.partial(
    pl.kernel,
    out_type=jax.ShapeDtypeStruct((BATCH, DP), jnp.float32),
    mesh=plsc.VectorSubcoreMesh(core_axis_name="c", subcore_axis_name="s"),
    scratch_types=[
        pltpu.VMEM((NCH * GPC, GW), jnp.int32),
        pltpu.VMEM((2, ROWS_PER_CHUNK, DP), jnp.float32),
        pltpu.VMEM((CB, DP), jnp.float32),
        pltpu.SemaphoreType.DMA,
    ],
    compiler_params=pltpu.CompilerParams(use_tc_tiling_on_sc=False),
)
def _sc_pool(proj_hbm, text_hbm, out_hbm, idx_v, rows_v, out_v, sem):
    _sc_body(proj_hbm, text_hbm, out_hbm, idx_v, rows_v, out_v, sem)


def kernel(text, emb_table, fc_w, fc_b):
    w_pad = jnp.zeros((DP, EMBED_DIM), jnp.float32).at[:OUTPUT_DIM].set(fc_w)
    b_pad = jnp.zeros((1, DP), jnp.float32).at[0, :OUTPUT_DIM].set(fc_b)
    proj = _project_table(emb_table, w_pad, b_pad)
    text2d = text.reshape(BATCH * SEQ // GW, GW).astype(jnp.int32)
    out = _sc_pool(proj, text2d)
    return out[:, :OUTPUT_DIM]


# speedup vs baseline: 1.1988x; 1.1988x over previous
import functools

import jax
import jax.numpy as jnp
from jax import lax
from jax.experimental import pallas as pl
from jax.experimental.pallas import tpu as pltpu
from jax.experimental.pallas import tpu_sc as plsc

VOCAB = 100000
EMBED_DIM = 64
OUTPUT_DIM = 2
BATCH = 4096
SEQ = 200

DP = 16
PACK = 128 // DP
NC, NS = 2, 16
NW = NC * NS
B_PER_W = BATCH // NW
CB = 16
NCH = B_PER_W // CB
GW = 128
GPC = CB * SEQ // GW
ROWS_PER_CHUNK = CB * SEQ
IDX_COLS = 128
IDX_ROWS = BATCH * SEQ // IDX_COLS
IDX_ROWS_PER_W = IDX_ROWS // NW
IDX_ROWS_PER_CHUNK = IDX_ROWS_PER_W // NCH


def _proj_body(emb_ref, w_ref, b_ref, out_ref):
    acc = jnp.dot(emb_ref[...], w_ref[...], preferred_element_type=jnp.float32)
    out_ref[...] = (acc + b_ref[...]) * (1.0 / SEQ)


def _project_table(emb_packed, w_blockdiag, b_tiled):
    blk = 640
    grid = pl.cdiv(VOCAB // PACK, blk)
    return pl.pallas_call(
        _proj_body,
        grid=(grid,),
        in_specs=[
            pl.BlockSpec((blk, PACK * EMBED_DIM), lambda i: (i, 0)),
            pl.BlockSpec((PACK * EMBED_DIM, DP * PACK), lambda i: (0, 0)),
            pl.BlockSpec((1, DP * PACK), lambda i: (0, 0)),
        ],
        out_specs=pl.BlockSpec((blk, DP * PACK), lambda i: (i, 0)),
        out_shape=jax.ShapeDtypeStruct((VOCAB // PACK, DP * PACK), jnp.float32),
    )(emb_packed, w_blockdiag, b_tiled)


def _sc_body(proj_hbm, text_hbm, out_hbm, idx_v, rows_v, out_v, sem):
    wid = lax.axis_index("s") * NC + lax.axis_index("c")

    def fire(c, slot):
        pltpu.sync_copy(
            text_hbm.at[pl.ds(wid * IDX_ROWS_PER_W + c * IDX_ROWS_PER_CHUNK,
                              IDX_ROWS_PER_CHUNK)],
            idx_v.at[slot])

        def issue(k, carry):
            pltpu.async_copy(
                proj_hbm.at[idx_v.at[slot, k]],
                rows_v.at[slot, pl.ds(k * GW, GW)],
                sem,
            )
            return carry

        lax.fori_loop(0, GPC, issue, 0)

    def drain(slot):
        pltpu.make_async_copy(
            proj_hbm.at[pl.ds(0, ROWS_PER_CHUNK)],
            rows_v.at[slot],
            sem,
        ).wait()

    def chunk_body(c, carry):
        slot = lax.rem(c, 2)
        drain(slot)

        @pl.when(c + 1 < NCH)
        def _():
            fire(c + 1, 1 - slot)

        for i in range(CB):
            base = i * SEQ

            def red_body(j, accs):
                o = base + 8 * j
                return tuple(accs[t] + rows_v[slot, o + t] for t in range(8))

            z = jnp.zeros((DP,), jnp.float32)
            accs = lax.fori_loop(0, SEQ // 8, red_body, (z,) * 8)
            s0 = (accs[0] + accs[1]) + (accs[2] + accs[3])
            s1 = (accs[4] + accs[5]) + (accs[6] + accs[7])
            out_v[i] = s0 + s1
        pltpu.sync_copy(out_v, out_hbm.at[pl.ds(wid * B_PER_W + c * CB, CB)])
        return carry

    fire(0, 0)
    lax.fori_loop(0, NCH, chunk_body, 0)


@functools.partial(
    pl.kernel,
    out_type=jax.ShapeDtypeStruct((BATCH, DP), jnp.float32),
    mesh=plsc.VectorSubcoreMesh(core_axis_name="c", subcore_axis_name="s"),
    scratch_types=[
        pltpu.VMEM((2, IDX_ROWS_PER_CHUNK, IDX_COLS), jnp.int32),
        pltpu.VMEM((2, ROWS_PER_CHUNK, DP), jnp.float32),
        pltpu.VMEM((CB, DP), jnp.float32),
        pltpu.SemaphoreType.DMA,
    ],
    compiler_params=pltpu.CompilerParams(use_tc_tiling_on_sc=False),
)
def _sc_pool(proj_hbm, text_hbm, out_hbm, idx_v, rows_v, out_v, sem):
    _sc_body(proj_hbm, text_hbm, out_hbm, idx_v, rows_v, out_v, sem)


def kernel(text, emb_table, fc_w, fc_b):
    wt_pad = jnp.zeros((EMBED_DIM, DP), jnp.float32).at[:, :OUTPUT_DIM].set(fc_w.T)
    w_bd = jnp.zeros((PACK * EMBED_DIM, DP * PACK), jnp.float32)
    for a in range(PACK):
        w_bd = w_bd.at[a * EMBED_DIM:(a + 1) * EMBED_DIM,
                       a * DP:(a + 1) * DP].set(wt_pad)
    b_pad = jnp.zeros((1, DP), jnp.float32).at[0, :OUTPUT_DIM].set(fc_b)
    b_tiled = jnp.tile(b_pad, (1, PACK))
    emb_packed = emb_table.reshape(VOCAB // PACK, PACK * EMBED_DIM)
    proj = _project_table(emb_packed, w_bd, b_tiled).reshape(VOCAB, DP)
    text2d = text.reshape(IDX_ROWS, IDX_COLS).astype(jnp.int32)
    out = _sc_pool(proj, text2d)
    return out[:, :OUTPUT_DIM]


# speedup vs baseline: 1.2690x; 1.0585x over previous
import functools

import jax
import jax.numpy as jnp
from jax import lax
from jax.experimental import pallas as pl
from jax.experimental.pallas import tpu as pltpu
from jax.experimental.pallas import tpu_sc as plsc

VOCAB = 100000
EMBED_DIM = 64
OUTPUT_DIM = 2
BATCH = 4096
SEQ = 200

DP = 16
PACK = 128 // DP
NC, NS = 2, 16
NW = NC * NS
B_PER_W = BATCH // NW
CB = 16
NCH = B_PER_W // CB
ROWS_PER_CHUNK = CB * SEQ
SEQ_PAD = 256
GW0 = 128
GW1 = SEQ - GW0


def _proj_body(emb_ref, w_ref, b_ref, out_ref):
    acc = jnp.dot(emb_ref[...], w_ref[...], preferred_element_type=jnp.float32)
    out_ref[...] = (acc + b_ref[...]) * (1.0 / SEQ)


def _project_table(emb_packed, w_blockdiag, b_tiled):
    blk = 2560
    grid = pl.cdiv(VOCAB // PACK, blk)
    return pl.pallas_call(
        _proj_body,
        grid=(grid,),
        in_specs=[
            pl.BlockSpec((blk, PACK * EMBED_DIM), lambda i: (i, 0)),
            pl.BlockSpec((PACK * EMBED_DIM, DP * PACK), lambda i: (0, 0)),
            pl.BlockSpec((1, DP * PACK), lambda i: (0, 0)),
        ],
        out_specs=pl.BlockSpec((blk, DP * PACK), lambda i: (i, 0)),
        out_shape=jax.ShapeDtypeStruct((VOCAB // PACK, DP * PACK), jnp.float32),
    )(emb_packed, w_blockdiag, b_tiled)


def _sc_body(proj_hbm, text_hbm, out_hbm, idx_v, rows_v, out_v, sem):
    wid = lax.axis_index("s") * NC + lax.axis_index("c")

    def fire(c, slot):
        pltpu.sync_copy(
            text_hbm.at[pl.ds(wid * B_PER_W + c * CB, CB)],
            idx_v.at[slot])

        def issue(r, carry):
            pltpu.async_copy(
                proj_hbm.at[idx_v.at[slot, r, pl.ds(0, GW0)]],
                rows_v.at[slot, pl.ds(r * SEQ, GW0)],
                sem,
            )
            pltpu.async_copy(
                proj_hbm.at[idx_v.at[slot, r, pl.ds(GW0, GW1)]],
                rows_v.at[slot, pl.ds(r * SEQ + GW0, GW1)],
                sem,
            )
            return carry

        lax.fori_loop(0, CB, issue, 0)

    def drain(slot):
        pltpu.make_async_copy(
            proj_hbm.at[pl.ds(0, ROWS_PER_CHUNK)],
            rows_v.at[slot],
            sem,
        ).wait()

    def chunk_body(c, carry):
        slot = lax.rem(c, 2)
        drain(slot)

        @pl.when(c + 1 < NCH)
        def _():
            fire(c + 1, 1 - slot)

        for i in range(CB):
            base = i * SEQ

            def red_body(j, accs):
                o = base + 8 * j
                return tuple(accs[t] + rows_v[slot, o + t] for t in range(8))

            z = jnp.zeros((DP,), jnp.float32)
            accs = lax.fori_loop(0, SEQ // 8, red_body, (z,) * 8)
            s0 = (accs[0] + accs[1]) + (accs[2] + accs[3])
            s1 = (accs[4] + accs[5]) + (accs[6] + accs[7])
            out_v[i] = s0 + s1
        pltpu.sync_copy(out_v, out_hbm.at[pl.ds(wid * B_PER_W + c * CB, CB)])
        return carry

    fire(0, 0)
    lax.fori_loop(0, NCH, chunk_body, 0)


@functools.partial(
    pl.kernel,
    out_type=jax.ShapeDtypeStruct((BATCH, DP), jnp.float32),
    mesh=plsc.VectorSubcoreMesh(core_axis_name="c", subcore_axis_name="s"),
    scratch_types=[
        pltpu.VMEM((2, CB, SEQ_PAD), jnp.int32),
        pltpu.VMEM((2, ROWS_PER_CHUNK, DP), jnp.float32),
        pltpu.VMEM((CB, DP), jnp.float32),
        pltpu.SemaphoreType.DMA,
    ],
    compiler_params=pltpu.CompilerParams(use_tc_tiling_on_sc=False),
)
def _sc_pool(proj_hbm, text_hbm, out_hbm, idx_v, rows_v, out_v, sem):
    _sc_body(proj_hbm, text_hbm, out_hbm, idx_v, rows_v, out_v, sem)


def kernel(text, emb_table, fc_w, fc_b):
    wt_pad = jnp.zeros((EMBED_DIM, DP), jnp.float32).at[:, :OUTPUT_DIM].set(fc_w.T)
    w_bd = jnp.zeros((PACK * EMBED_DIM, DP * PACK), jnp.float32)
    for a in range(PACK):
        w_bd = w_bd.at[a * EMBED_DIM:(a + 1) * EMBED_DIM,
                       a * DP:(a + 1) * DP].set(wt_pad)
    b_pad = jnp.zeros((1, DP), jnp.float32).at[0, :OUTPUT_DIM].set(fc_b)
    b_tiled = jnp.tile(b_pad, (1, PACK))
    emb_packed = emb_table.reshape(VOCAB // PACK, PACK * EMBED_DIM)
    proj = _project_table(emb_packed, w_bd, b_tiled).reshape(VOCAB, DP)
    text_p = jnp.pad(text.astype(jnp.int32), ((0, 0), (0, SEQ_PAD - SEQ)))
    out = _sc_pool(proj, text_p)
    return out[:, :OUTPUT_DIM]


# speedup vs baseline: 1.2715x; 1.0020x over previous
import functools

import jax
import jax.numpy as jnp
from jax import lax
from jax.experimental import pallas as pl
from jax.experimental.pallas import tpu as pltpu
from jax.experimental.pallas import tpu_sc as plsc

VOCAB = 100000
EMBED_DIM = 64
OUTPUT_DIM = 2
BATCH = 4096
SEQ = 200

DP = 16
PACK = 128 // DP
NC, NS = 2, 16
NW = NC * NS
B_PER_W = BATCH // NW
CB = 16
NCH = B_PER_W // CB
ROWS_PER_CHUNK = CB * SEQ
SEQ_PAD = 256
GW0 = 128
GW1 = SEQ - GW0


def _proj_body(emb_ref, w_ref, b_ref, out_ref):
    wt = w_ref[...]
    wcols = jnp.concatenate([wt] * PACK, axis=1)
    wtile = jnp.concatenate([wcols] * PACK, axis=0)
    ra = lax.broadcasted_iota(jnp.int32, wtile.shape, 0) // EMBED_DIM
    ca = lax.broadcasted_iota(jnp.int32, wtile.shape, 1) // DP
    w_bd = jnp.where(ra == ca, wtile, 0.0)
    bt = jnp.concatenate([b_ref[...]] * PACK, axis=1)
    acc = jnp.dot(emb_ref[...], w_bd, preferred_element_type=jnp.float32)
    out_ref[...] = (acc + bt) * (1.0 / SEQ)


def _project_table(emb_packed, w_blockdiag, b_tiled):
    blk = 2560
    grid = pl.cdiv(VOCAB // PACK, blk)
    return pl.pallas_call(
        _proj_body,
        grid=(grid,),
        in_specs=[
            pl.BlockSpec((blk, PACK * EMBED_DIM), lambda i: (i, 0)),
            pl.BlockSpec((EMBED_DIM, DP), lambda i: (0, 0)),
            pl.BlockSpec((1, DP), lambda i: (0, 0)),
        ],
        out_specs=pl.BlockSpec((blk, DP * PACK), lambda i: (i, 0)),
        out_shape=jax.ShapeDtypeStruct((VOCAB // PACK, DP * PACK), jnp.float32),
    )(emb_packed, w_blockdiag, b_tiled)


def _sc_body(proj_hbm, text_hbm, out_hbm, idx_v, rows_v, out_v, sem):
    wid = lax.axis_index("s") * NC + lax.axis_index("c")

    def fire(c, slot):
        pltpu.sync_copy(
            text_hbm.at[pl.ds(wid * B_PER_W + c * CB, CB)],
            idx_v.at[slot])

        def issue(r, carry):
            pltpu.async_copy(
                proj_hbm.at[idx_v.at[slot, r, pl.ds(0, GW0)]],
                rows_v.at[slot, pl.ds(r * SEQ, GW0)],
                sem,
            )
            pltpu.async_copy(
                proj_hbm.at[idx_v.at[slot, r, pl.ds(GW0, GW1)]],
                rows_v.at[slot, pl.ds(r * SEQ + GW0, GW1)],
                sem,
            )
            return carry

        lax.fori_loop(0, CB, issue, 0)

    def drain(slot):
        pltpu.make_async_copy(
            proj_hbm.at[pl.ds(0, ROWS_PER_CHUNK)],
            rows_v.at[slot],
            sem,
        ).wait()

    def chunk_body(c, carry):
        slot = lax.rem(c, 2)
        drain(slot)

        @pl.when(c + 1 < NCH)
        def _():
            fire(c + 1, 1 - slot)

        for i in range(CB):
            base = i * SEQ

            def red_body(j, accs):
                o = base + 8 * j
                return tuple(accs[t] + rows_v[slot, o + t] for t in range(8))

            z = jnp.zeros((DP,), jnp.float32)
            accs = lax.fori_loop(0, SEQ // 8, red_body, (z,) * 8)
            s0 = (accs[0] + accs[1]) + (accs[2] + accs[3])
            s1 = (accs[4] + accs[5]) + (accs[6] + accs[7])
            out_v[i] = s0 + s1
        pltpu.sync_copy(out_v, out_hbm.at[pl.ds(wid * B_PER_W + c * CB, CB)])
        return carry

    fire(0, 0)
    lax.fori_loop(0, NCH, chunk_body, 0)


@functools.partial(
    pl.kernel,
    out_type=jax.ShapeDtypeStruct((BATCH, DP), jnp.float32),
    mesh=plsc.VectorSubcoreMesh(core_axis_name="c", subcore_axis_name="s"),
    scratch_types=[
        pltpu.VMEM((2, CB, SEQ_PAD), jnp.int32),
        pltpu.VMEM((2, ROWS_PER_CHUNK, DP), jnp.float32),
        pltpu.VMEM((CB, DP), jnp.float32),
        pltpu.SemaphoreType.DMA,
    ],
    compiler_params=pltpu.CompilerParams(use_tc_tiling_on_sc=False),
)
def _sc_pool(proj_hbm, text_hbm, out_hbm, idx_v, rows_v, out_v, sem):
    _sc_body(proj_hbm, text_hbm, out_hbm, idx_v, rows_v, out_v, sem)


def kernel(text, emb_table, fc_w, fc_b):
    wt_pad = jnp.zeros((EMBED_DIM, DP), jnp.float32).at[:, :OUTPUT_DIM].set(fc_w.T)
    b_pad = jnp.zeros((1, DP), jnp.float32).at[0, :OUTPUT_DIM].set(fc_b)
    emb_packed = emb_table.reshape(VOCAB // PACK, PACK * EMBED_DIM)
    proj = _project_table(emb_packed, wt_pad, b_pad).reshape(VOCAB, DP)
    text_p = jnp.pad(text.astype(jnp.int32), ((0, 0), (0, SEQ_PAD - SEQ)))
    out = _sc_pool(proj, text_p)
    return out[:, :OUTPUT_DIM]


# speedup vs baseline: 1.2751x; 1.0029x over previous
import functools

import jax
import jax.numpy as jnp
from jax import lax
from jax.experimental import pallas as pl
from jax.experimental.pallas import tpu as pltpu
from jax.experimental.pallas import tpu_sc as plsc

VOCAB = 100000
EMBED_DIM = 64
OUTPUT_DIM = 2
BATCH = 4096
SEQ = 200

DP = 16
PACK = 128 // DP
NC, NS = 2, 16
NW = NC * NS
B_PER_W = BATCH // NW
CB = 16
NCH = B_PER_W // CB
ROWS_PER_CHUNK = CB * SEQ
GW0 = 128
GW1 = SEQ - GW0


def _proj_body(emb_ref, w_ref, b_ref, out_ref):
    wt = w_ref[...]
    wcols = jnp.concatenate([wt] * PACK, axis=1)
    wtile = jnp.concatenate([wcols] * PACK, axis=0)
    ra = lax.broadcasted_iota(jnp.int32, wtile.shape, 0) // EMBED_DIM
    ca = lax.broadcasted_iota(jnp.int32, wtile.shape, 1) // DP
    w_bd = jnp.where(ra == ca, wtile, 0.0)
    bt = jnp.concatenate([b_ref[...]] * PACK, axis=1)
    acc = jnp.dot(emb_ref[...], w_bd, preferred_element_type=jnp.float32)
    out_ref[...] = (acc + bt) * (1.0 / SEQ)


def _project_table(emb_packed, w_blockdiag, b_tiled):
    blk = 2560
    grid = pl.cdiv(VOCAB // PACK, blk)
    return pl.pallas_call(
        _proj_body,
        grid=(grid,),
        in_specs=[
            pl.BlockSpec((blk, PACK * EMBED_DIM), lambda i: (i, 0)),
            pl.BlockSpec((EMBED_DIM, DP), lambda i: (0, 0)),
            pl.BlockSpec((1, DP), lambda i: (0, 0)),
        ],
        out_specs=pl.BlockSpec((blk, DP * PACK), lambda i: (i, 0)),
        out_shape=jax.ShapeDtypeStruct((VOCAB // PACK, DP * PACK), jnp.float32),
    )(emb_packed, w_blockdiag, b_tiled)


def _sc_body(proj_hbm, text_hbm, out_hbm, idx_v, rows_v, out_v, sem):
    wid = lax.axis_index("s") * NC + lax.axis_index("c")

    def fire(c, slot):
        pltpu.sync_copy(
            text_hbm.at[pl.ds(wid * B_PER_W + c * CB, CB)],
            idx_v.at[slot])

        def issue(r, carry):
            pltpu.async_copy(
                proj_hbm.at[idx_v.at[slot, r, pl.ds(0, GW0)]],
                rows_v.at[slot, pl.ds(r * SEQ, GW0)],
                sem,
            )
            pltpu.async_copy(
                proj_hbm.at[idx_v.at[slot, r, pl.ds(GW0, GW1)]],
                rows_v.at[slot, pl.ds(r * SEQ + GW0, GW1)],
                sem,
            )
            return carry

        lax.fori_loop(0, CB, issue, 0)

    def drain(slot):
        pltpu.make_async_copy(
            proj_hbm.at[pl.ds(0, ROWS_PER_CHUNK)],
            rows_v.at[slot],
            sem,
        ).wait()

    def chunk_body(c, carry):
        slot = lax.rem(c, 2)
        drain(slot)

        @pl.when(c + 1 < NCH)
        def _():
            fire(c + 1, 1 - slot)

        for i in range(CB):
            base = i * SEQ

            def red_body(j, accs):
                o = base + 8 * j
                return tuple(accs[t] + rows_v[slot, o + t] for t in range(8))

            z = jnp.zeros((DP,), jnp.float32)
            accs = lax.fori_loop(0, SEQ // 8, red_body, (z,) * 8)
            s0 = (accs[0] + accs[1]) + (accs[2] + accs[3])
            s1 = (accs[4] + accs[5]) + (accs[6] + accs[7])
            out_v[i] = s0 + s1
        pltpu.sync_copy(out_v, out_hbm.at[pl.ds(wid * B_PER_W + c * CB, CB)])
        return carry

    fire(0, 0)
    lax.fori_loop(0, NCH, chunk_body, 0)


@functools.partial(
    pl.kernel,
    out_type=jax.ShapeDtypeStruct((BATCH, DP), jnp.float32),
    mesh=plsc.VectorSubcoreMesh(core_axis_name="c", subcore_axis_name="s"),
    scratch_types=[
        pltpu.VMEM((2, CB, SEQ), jnp.int32),
        pltpu.VMEM((2, ROWS_PER_CHUNK, DP), jnp.float32),
        pltpu.VMEM((CB, DP), jnp.float32),
        pltpu.SemaphoreType.DMA,
    ],
    compiler_params=pltpu.CompilerParams(use_tc_tiling_on_sc=False),
)
def _sc_pool(proj_hbm, text_hbm, out_hbm, idx_v, rows_v, out_v, sem):
    _sc_body(proj_hbm, text_hbm, out_hbm, idx_v, rows_v, out_v, sem)


def kernel(text, emb_table, fc_w, fc_b):
    wt_pad = jnp.zeros((EMBED_DIM, DP), jnp.float32).at[:, :OUTPUT_DIM].set(fc_w.T)
    b_pad = jnp.zeros((1, DP), jnp.float32).at[0, :OUTPUT_DIM].set(fc_b)
    emb_packed = emb_table.reshape(VOCAB // PACK, PACK * EMBED_DIM)
    proj = _project_table(emb_packed, wt_pad, b_pad).reshape(VOCAB, DP)
    out = _sc_pool(proj, text.astype(jnp.int32))
    return out[:, :OUTPUT_DIM]


# speedup vs baseline: 1.2807x; 1.0044x over previous
import functools

import jax
import jax.numpy as jnp
from jax import lax
from jax.experimental import pallas as pl
from jax.experimental.pallas import tpu as pltpu
from jax.experimental.pallas import tpu_sc as plsc

VOCAB = 100000
EMBED_DIM = 64
OUTPUT_DIM = 2
BATCH = 4096
SEQ = 200

DP = 16
PACK = 128 // DP
NC, NS = 2, 16
NW = NC * NS
B_PER_W = BATCH // NW
CB = 16
NCH = B_PER_W // CB
ROWS_PER_CHUNK = CB * SEQ
GW = 128
GPC = ROWS_PER_CHUNK // GW
IDX_ROWS = BATCH * SEQ // GW
IDX_PER_W = IDX_ROWS // NW
IDX_PER_CHUNK = IDX_PER_W // NCH


def _proj_body(emb_ref, w_ref, b_ref, out_ref):
    wt = w_ref[...]
    wcols = jnp.concatenate([wt] * PACK, axis=1)
    wtile = jnp.concatenate([wcols] * PACK, axis=0)
    ra = lax.broadcasted_iota(jnp.int32, wtile.shape, 0) // EMBED_DIM
    ca = lax.broadcasted_iota(jnp.int32, wtile.shape, 1) // DP
    w_bd = jnp.where(ra == ca, wtile, 0.0)
    bt = jnp.concatenate([b_ref[...]] * PACK, axis=1)
    acc = jnp.dot(emb_ref[...], w_bd, preferred_element_type=jnp.float32)
    out_ref[...] = (acc + bt) * (1.0 / SEQ)


def _project_table(emb_packed, w_blockdiag, b_tiled):
    blk = 2560
    grid = pl.cdiv(VOCAB // PACK, blk)
    return pl.pallas_call(
        _proj_body,
        grid=(grid,),
        in_specs=[
            pl.BlockSpec((blk, PACK * EMBED_DIM), lambda i: (i, 0)),
            pl.BlockSpec((EMBED_DIM, DP), lambda i: (0, 0)),
            pl.BlockSpec((1, DP), lambda i: (0, 0)),
        ],
        out_specs=pl.BlockSpec((blk, DP * PACK), lambda i: (i, 0)),
        out_shape=jax.ShapeDtypeStruct((VOCAB // PACK, DP * PACK), jnp.float32),
    )(emb_packed, w_blockdiag, b_tiled)


def _sc_body(proj_hbm, text_hbm, out_hbm, idx_v, rows_v, out_v, sem):
    wid = lax.axis_index("s") * NC + lax.axis_index("c")

    def fire(c, slot):
        pltpu.sync_copy(
            text_hbm.at[pl.ds(wid * IDX_PER_W + c * IDX_PER_CHUNK, IDX_PER_CHUNK)],
            idx_v.at[slot])

        def issue(k, carry):
            pltpu.async_copy(
                proj_hbm.at[idx_v.at[slot, k]],
                rows_v.at[slot, pl.ds(k * GW, GW)],
                sem,
            )
            return carry

        lax.fori_loop(0, GPC, issue, 0)

    def drain(slot):
        pltpu.make_async_copy(
            proj_hbm.at[pl.ds(0, ROWS_PER_CHUNK)],
            rows_v.at[slot],
            sem,
        ).wait()

    def chunk_body(c, carry):
        slot = lax.rem(c, 2)
        drain(slot)

        @pl.when(c + 1 < NCH)
        def _():
            fire(c + 1, 1 - slot)

        for i in range(CB):
            base = i * SEQ

            def red_body(j, accs):
                o = base + 8 * j
                return tuple(accs[t] + rows_v[slot, o + t] for t in range(8))

            z = jnp.zeros((DP,), jnp.float32)
            accs = lax.fori_loop(0, SEQ // 8, red_body, (z,) * 8)
            s0 = (accs[0] + accs[1]) + (accs[2] + accs[3])
            s1 = (accs[4] + accs[5]) + (accs[6] + accs[7])
            out_v[i] = s0 + s1
        pltpu.sync_copy(out_v, out_hbm.at[pl.ds(wid * B_PER_W + c * CB, CB)])
        return carry

    fire(0, 0)
    lax.fori_loop(0, NCH, chunk_body, 0)


@functools.partial(
    pl.kernel,
    out_type=jax.ShapeDtypeStruct((BATCH, DP), jnp.float32),
    mesh=plsc.VectorSubcoreMesh(core_axis_name="c", subcore_axis_name="s"),
    scratch_types=[
        pltpu.VMEM((2, IDX_PER_CHUNK, GW), jnp.int32),
        pltpu.VMEM((2, ROWS_PER_CHUNK, DP), jnp.float32),
        pltpu.VMEM((CB, DP), jnp.float32),
        pltpu.SemaphoreType.DMA,
    ],
    compiler_params=pltpu.CompilerParams(use_tc_tiling_on_sc=False),
)
def _sc_pool(proj_hbm, text_hbm, out_hbm, idx_v, rows_v, out_v, sem):
    _sc_body(proj_hbm, text_hbm, out_hbm, idx_v, rows_v, out_v, sem)


def kernel(text, emb_table, fc_w, fc_b):
    wt_pad = jnp.zeros((EMBED_DIM, DP), jnp.float32).at[:, :OUTPUT_DIM].set(fc_w.T)
    b_pad = jnp.zeros((1, DP), jnp.float32).at[0, :OUTPUT_DIM].set(fc_b)
    emb_packed = emb_table.reshape(VOCAB // PACK, PACK * EMBED_DIM)
    proj = _project_table(emb_packed, wt_pad, b_pad).reshape(VOCAB, DP)
    text2d = text.reshape(IDX_ROWS, GW).astype(jnp.int32)
    out = _sc_pool(proj, text2d)
    return out[:, :OUTPUT_DIM]


# speedup vs baseline: 1.2818x; 1.0008x over previous
import functools

import jax
import jax.numpy as jnp
from jax import lax
from jax.experimental import pallas as pl
from jax.experimental.pallas import tpu as pltpu
from jax.experimental.pallas import tpu_sc as plsc

VOCAB = 100000
EMBED_DIM = 64
OUTPUT_DIM = 2
BATCH = 4096
SEQ = 200

DP = 16
PACK = 128 // DP
NC, NS = 2, 16
NW = NC * NS
B_PER_W = BATCH // NW
CB = 16
NCH = B_PER_W // CB
ROWS_PER_CHUNK = CB * SEQ
GW = 128
GPC = ROWS_PER_CHUNK // GW
IDX_ROWS = BATCH * SEQ // GW
IDX_PER_W = IDX_ROWS // NW
IDX_PER_CHUNK = IDX_PER_W // NCH


def _proj_body(emb_ref, w_ref, b_ref, out_ref):
    wt = w_ref[...]
    wcols = jnp.concatenate([wt] * PACK, axis=1)
    wtile = jnp.concatenate([wcols] * PACK, axis=0)
    ra = lax.broadcasted_iota(jnp.int32, wtile.shape, 0) // EMBED_DIM
    ca = lax.broadcasted_iota(jnp.int32, wtile.shape, 1) // DP
    w_bd = jnp.where(ra == ca, wtile, 0.0)
    bt = jnp.concatenate([b_ref[...]] * PACK, axis=1)
    acc = jnp.dot(emb_ref[...], w_bd, preferred_element_type=jnp.float32)
    out_ref[...] = (acc + bt) * (1.0 / SEQ)


def _project_table(emb_packed, w_blockdiag, b_tiled):
    blk = 2560
    grid = pl.cdiv(VOCAB // PACK, blk)
    return pl.pallas_call(
        _proj_body,
        grid=(grid,),
        in_specs=[
            pl.BlockSpec((blk, PACK * EMBED_DIM), lambda i: (i, 0)),
            pl.BlockSpec((EMBED_DIM, DP), lambda i: (0, 0)),
            pl.BlockSpec((1, DP), lambda i: (0, 0)),
        ],
        out_specs=pl.BlockSpec((blk, DP * PACK), lambda i: (i, 0)),
        out_shape=jax.ShapeDtypeStruct((VOCAB // PACK, DP * PACK), jnp.float32),
        compiler_params=pltpu.CompilerParams(
            allow_input_fusion=[True, False, False]),
    )(emb_packed, w_blockdiag, b_tiled)


def _sc_body(proj_hbm, text_hbm, out_hbm, idx_v, rows_v, out_v, sem):
    wid = lax.axis_index("s") * NC + lax.axis_index("c")

    def fire(c, slot):
        pltpu.sync_copy(
            text_hbm.at[pl.ds(wid * IDX_PER_W + c * IDX_PER_CHUNK, IDX_PER_CHUNK)],
            idx_v.at[slot])

        def issue(k, carry):
            pltpu.async_copy(
                proj_hbm.at[idx_v.at[slot, k]],
                rows_v.at[slot, pl.ds(k * GW, GW)],
                sem,
            )
            return carry

        lax.fori_loop(0, GPC, issue, 0)

    def drain(slot):
        pltpu.make_async_copy(
            proj_hbm.at[pl.ds(0, ROWS_PER_CHUNK)],
            rows_v.at[slot],
            sem,
        ).wait()

    def chunk_body(c, carry):
        slot = lax.rem(c, 2)
        drain(slot)

        @pl.when(c + 1 < NCH)
        def _():
            fire(c + 1, 1 - slot)

        for i in range(CB):
            base = i * SEQ

            def red_body(j, accs):
                o = base + 8 * j
                return tuple(accs[t] + rows_v[slot, o + t] for t in range(8))

            z = jnp.zeros((DP,), jnp.float32)
            accs = lax.fori_loop(0, SEQ // 8, red_body, (z,) * 8)
            s0 = (accs[0] + accs[1]) + (accs[2] + accs[3])
            s1 = (accs[4] + accs[5]) + (accs[6] + accs[7])
            out_v[i] = s0 + s1
        pltpu.sync_copy(out_v, out_hbm.at[pl.ds(wid * B_PER_W + c * CB, CB)])
        return carry

    fire(0, 0)
    lax.fori_loop(0, NCH, chunk_body, 0)


@functools.partial(
    pl.kernel,
    out_type=jax.ShapeDtypeStruct((BATCH, DP), jnp.float32),
    mesh=plsc.VectorSubcoreMesh(core_axis_name="c", subcore_axis_name="s"),
    scratch_types=[
        pltpu.VMEM((2, IDX_PER_CHUNK, GW), jnp.int32),
        pltpu.VMEM((2, ROWS_PER_CHUNK, DP), jnp.float32),
        pltpu.VMEM((CB, DP), jnp.float32),
        pltpu.SemaphoreType.DMA,
    ],
    compiler_params=pltpu.CompilerParams(use_tc_tiling_on_sc=False),
)
def _sc_pool(proj_hbm, text_hbm, out_hbm, idx_v, rows_v, out_v, sem):
    _sc_body(proj_hbm, text_hbm, out_hbm, idx_v, rows_v, out_v, sem)


def kernel(text, emb_table, fc_w, fc_b):
    wt_pad = jnp.zeros((EMBED_DIM, DP), jnp.float32).at[:, :OUTPUT_DIM].set(fc_w.T)
    b_pad = jnp.zeros((1, DP), jnp.float32).at[0, :OUTPUT_DIM].set(fc_b)
    emb_packed = emb_table.reshape(VOCAB // PACK, PACK * EMBED_DIM)
    proj = _project_table(emb_packed, wt_pad, b_pad).reshape(VOCAB, DP)
    text2d = text.reshape(IDX_ROWS, GW).astype(jnp.int32)
    out = _sc_pool(proj, text2d)
    return out[:, :OUTPUT_DIM]


# speedup vs baseline: 1.3293x; 1.0371x over previous
import functools

import jax
import jax.numpy as jnp
from jax import lax
from jax.experimental import pallas as pl
from jax.experimental.pallas import tpu as pltpu
from jax.experimental.pallas import tpu_sc as plsc

VOCAB = 100000
EMBED_DIM = 64
OUTPUT_DIM = 2
BATCH = 4096
SEQ = 200

DP = 16
PACK = 128 // DP
NC, NS = 2, 16
NW = NC * NS
B_PER_W = BATCH // NW
CB = 8
NCH = B_PER_W // CB
ROWS_PER_CHUNK = CB * SEQ
GW = 100
GPC = ROWS_PER_CHUNK // GW
IDX_ROWS = BATCH * SEQ // GW
IDX_PER_W = IDX_ROWS // NW
IDX_PER_CHUNK = IDX_PER_W // NCH
NSLOT = 4


def _proj_body(emb_ref, w_ref, b_ref, out_ref):
    wt = w_ref[...]
    wcols = jnp.concatenate([wt] * PACK, axis=1)
    wtile = jnp.concatenate([wcols] * PACK, axis=0)
    ra = lax.broadcasted_iota(jnp.int32, wtile.shape, 0) // EMBED_DIM
    ca = lax.broadcasted_iota(jnp.int32, wtile.shape, 1) // DP
    w_bd = jnp.where(ra == ca, wtile, 0.0)
    bt = jnp.concatenate([b_ref[...]] * PACK, axis=1)
    acc = jnp.dot(emb_ref[...], w_bd, preferred_element_type=jnp.float32)
    out_ref[...] = (acc + bt) * (1.0 / SEQ)


def _project_table(emb_packed, w_blockdiag, b_tiled):
    blk = 2560
    grid = pl.cdiv(VOCAB // PACK, blk)
    return pl.pallas_call(
        _proj_body,
        grid=(grid,),
        in_specs=[
            pl.BlockSpec((blk, PACK * EMBED_DIM), lambda i: (i, 0)),
            pl.BlockSpec((EMBED_DIM, DP), lambda i: (0, 0)),
            pl.BlockSpec((1, DP), lambda i: (0, 0)),
        ],
        out_specs=pl.BlockSpec((blk, DP * PACK), lambda i: (i, 0)),
        out_shape=jax.ShapeDtypeStruct((VOCAB // PACK, DP * PACK), jnp.float32),
        compiler_params=pltpu.CompilerParams(
            allow_input_fusion=[True, False, False]),
    )(emb_packed, w_blockdiag, b_tiled)


def _sc_body(proj_hbm, text_hbm, out_hbm, idx_v, rows_v, out_v, sem):
    wid = lax.axis_index("s") * NC + lax.axis_index("c")

    def fire(c, slot):
        pltpu.sync_copy(
            text_hbm.at[pl.ds(wid * IDX_PER_W + c * IDX_PER_CHUNK, IDX_PER_CHUNK)],
            idx_v.at[slot])

        def issue(k, carry):
            pltpu.async_copy(
                proj_hbm.at[idx_v.at[slot, k]],
                rows_v.at[slot, pl.ds(k * GW, GW)],
                sem.at[slot],
            )
            return carry

        lax.fori_loop(0, GPC, issue, 0)

    def drain(slot):
        pltpu.make_async_copy(
            proj_hbm.at[pl.ds(0, ROWS_PER_CHUNK)],
            rows_v.at[slot],
            sem.at[slot],
        ).wait()

    def chunk_body(c, carry):
        slot = lax.rem(c, NSLOT)
        drain(slot)

        @pl.when(c + NSLOT - 1 < NCH)
        def _():
            fire(c + NSLOT - 1, lax.rem(c + NSLOT - 1, NSLOT))

        for i in range(CB):
            base = i * SEQ

            def red_body(j, accs):
                o = base + 8 * j
                return tuple(accs[t] + rows_v[slot, o + t] for t in range(8))

            z = jnp.zeros((DP,), jnp.float32)
            accs = lax.fori_loop(0, SEQ // 8, red_body, (z,) * 8)
            s0 = (accs[0] + accs[1]) + (accs[2] + accs[3])
            s1 = (accs[4] + accs[5]) + (accs[6] + accs[7])
            out_v[i] = s0 + s1
        pltpu.sync_copy(out_v, out_hbm.at[pl.ds(wid * B_PER_W + c * CB, CB)])
        return carry

    for p in range(NSLOT - 1):
        fire(p, p)
    lax.fori_loop(0, NCH, chunk_body, 0)


@functools.partial(
    pl.kernel,
    out_type=jax.ShapeDtypeStruct((BATCH, DP), jnp.float32),
    mesh=plsc.VectorSubcoreMesh(core_axis_name="c", subcore_axis_name="s"),
    scratch_types=[
        pltpu.VMEM((NSLOT, IDX_PER_CHUNK, GW), jnp.int32),
        pltpu.VMEM((NSLOT, ROWS_PER_CHUNK, DP), jnp.float32),
        pltpu.VMEM((CB, DP), jnp.float32),
        pltpu.SemaphoreType.DMA((NSLOT,)),
    ],
    compiler_params=pltpu.CompilerParams(use_tc_tiling_on_sc=False),
)
def _sc_pool(proj_hbm, text_hbm, out_hbm, idx_v, rows_v, out_v, sem):
    _sc_body(proj_hbm, text_hbm, out_hbm, idx_v, rows_v, out_v, sem)


def kernel(text, emb_table, fc_w, fc_b):
    wt_pad = jnp.zeros((EMBED_DIM, DP), jnp.float32).at[:, :OUTPUT_DIM].set(fc_w.T)
    b_pad = jnp.zeros((1, DP), jnp.float32).at[0, :OUTPUT_DIM].set(fc_b)
    emb_packed = emb_table.reshape(VOCAB // PACK, PACK * EMBED_DIM)
    proj = _project_table(emb_packed, wt_pad, b_pad).reshape(VOCAB, DP)
    text2d = text.reshape(IDX_ROWS, GW).astype(jnp.int32)
    out = _sc_pool(proj, text2d)
    return out[:, :OUTPUT_DIM]
